# Initial kernel scaffold; baseline (speedup 1.0000x reference)
import jax
import jax.numpy as jnp
from jax.experimental import pallas as pl


def kernel(edge_index, users_emb, items_emb):
    raise NotImplementedError("write your pallas kernel here")


# speedup vs baseline: 5.5923x; 5.5923x over previous
import functools

import jax
import jax.numpy as jnp
from jax import lax
from jax.experimental import pallas as pl
from jax.experimental.pallas import tpu as pltpu
from jax.experimental.pallas import tpu_sc as plsc

NUM_USERS = 10000
NUM_ITEMS = 40000
NV = NUM_USERS + NUM_ITEMS
E = 800000
D = 64
NC, NS = 2, 16
HALF = NV // NC

SCH = 8
ER = 6256
E_PAD = ER * 128
NSCH = ER // SCH
ACC_ROWS = 25088
ZR = ACC_ROWS // NS
TAIL = HALF - 15 * ZR
DH = 32
DEG_PAD = 50048
DZ = DEG_PAD // NS

_sc_mesh = plsc.VectorSubcoreMesh(core_axis_name="c", subcore_axis_name="s")
_sc_params = pltpu.CompilerParams(use_tc_tiling_on_sc=False, internal_scratch_in_bytes=0)


@functools.partial(
    pl.kernel,
    out_type=jax.ShapeDtypeStruct((DEG_PAD,), jnp.float32),
    mesh=_sc_mesh,
    compiler_params=_sc_params,
    scratch_types=[
        [pltpu.VMEM((128,), jnp.int32) for _ in range(SCH)],
        pltpu.VMEM((128,), jnp.float32),
        pltpu.VMEM((DZ,), jnp.float32),
        pltpu.VMEM_SHARED((DEG_PAD,), jnp.float32),
        pltpu.SemaphoreType.DMA,
    ],
)
def _deg_sc(row1_h, zd_h, deg_h, rb, obuf, dbuf, dacc, sem):
    c = lax.axis_index("c")
    s = lax.axis_index("s")

    @pl.when(c == 0)
    def _():
        for m in range(8):
            obuf[pl.ds(m * 16, 16)] = jnp.full((16,), 1.0, jnp.float32)
        pltpu.sync_copy(zd_h, dbuf)
        pltpu.sync_copy(dbuf, dacc.at[pl.ds(s * DZ, DZ)])
        plsc.subcore_barrier()

        def chunk(i, carry):
            j = i * NS + s

            @pl.when(j < NSCH)
            def _():
                base = j * SCH * 128
                loads = [
                    pltpu.async_copy(
                        row1_h.at[pl.ds(base + m * 128, 128)], rb[m], sem
                    )
                    for m in range(SCH)
                ]
                for ld in loads:
                    ld.wait()
                for m in range(SCH):
                    pltpu.sync_copy(obuf, dacc.at[rb[m]], add=True)

            return carry

        lax.fori_loop(0, (NSCH + NS - 1) // NS, chunk, 0)
        plsc.subcore_barrier()
        pltpu.sync_copy(dacc.at[pl.ds(s * DZ, DZ)], dbuf)
        pltpu.sync_copy(dbuf, deg_h.at[pl.ds(s * DZ, DZ)])


@functools.partial(
    pl.kernel,
    out_type=jax.ShapeDtypeStruct((NV, DH), jnp.float32),
    mesh=_sc_mesh,
    compiler_params=_sc_params,
    scratch_types=[
        [pltpu.VMEM((128,), jnp.int32) for _ in range(SCH)],
        [pltpu.VMEM((128,), jnp.int32) for _ in range(SCH)],
        pltpu.VMEM((SCH * 128, DH), jnp.float32),
        pltpu.VMEM_SHARED((ACC_ROWS, DH), jnp.float32),
        pltpu.SemaphoreType.DMA,
        pltpu.SemaphoreType.DMA,
        pltpu.SemaphoreType.DMA,
    ],
)
def _spmm_sc(col1_h, lrow_h, s_h, zeros_h, out_h, cb, lb, gbig, acc, isem, gsem, ssem):
    c = lax.axis_index("c")
    s = lax.axis_index("s")

    pltpu.sync_copy(zeros_h, gbig)
    pltpu.sync_copy(gbig, acc.at[pl.ds(s * ZR, 1024)])
    pltpu.sync_copy(gbig.at[pl.ds(0, ZR - 1024)], acc.at[pl.ds(s * ZR + 1024, ZR - 1024)])
    plsc.subcore_barrier()

    lbase = c * E_PAD

    def chunk(i, carry):
        j = i * NS + s

        @pl.when(j < NSCH)
        def _():
            base = j * SCH * 128
            loads = [
                pltpu.async_copy(col1_h.at[pl.ds(base + m * 128, 128)], cb[m], isem)
                for m in range(SCH)
            ] + [
                pltpu.async_copy(
                    lrow_h.at[pl.ds(lbase + base + m * 128, 128)], lb[m], isem
                )
                for m in range(SCH)
            ]
            for ld in loads:
                ld.wait()
            gathers = [
                pltpu.async_copy(
                    s_h.at[cb[m]], gbig.at[pl.ds(m * 128, 128)], gsem
                )
                for m in range(SCH)
            ]
            for g in gathers:
                g.wait()
            scatters = [
                pltpu.async_copy(
                    gbig.at[pl.ds(m * 128, 128)], acc.at[lb[m]], ssem, add=True
                )
                for m in range(SCH)
            ]
            for sc in scatters:
                sc.wait()

        return carry

    lax.fori_loop(0, (NSCH + NS - 1) // NS, chunk, 0)
    plsc.subcore_barrier()

    @pl.when(s < NS - 1)
    def _():
        pltpu.sync_copy(acc.at[pl.ds(s * ZR, 1024)], gbig)
        pltpu.sync_copy(gbig, out_h.at[pl.ds(c * HALF + s * ZR, 1024)])
        pltpu.sync_copy(acc.at[pl.ds(s * ZR + 1024, ZR - 1024)], gbig.at[pl.ds(0, ZR - 1024)])
        pltpu.sync_copy(
            gbig.at[pl.ds(0, ZR - 1024)],
            out_h.at[pl.ds(c * HALF + s * ZR + 1024, ZR - 1024)],
        )

    @pl.when(s == NS - 1)
    def _():
        pltpu.sync_copy(acc.at[pl.ds((NS - 1) * ZR, 1024)], gbig)
        pltpu.sync_copy(gbig, out_h.at[pl.ds(c * HALF + (NS - 1) * ZR, 1024)])
        pltpu.sync_copy(
            acc.at[pl.ds((NS - 1) * ZR + 1024, TAIL - 1024)],
            gbig.at[pl.ds(0, TAIL - 1024)],
        )
        pltpu.sync_copy(
            gbig.at[pl.ds(0, TAIL - 1024)],
            out_h.at[pl.ds(c * HALF + (NS - 1) * ZR + 1024, TAIL - 1024)],
        )


_R = 5000


def _prep_tc(deg2d, emb0):
    def body(dref, eref, s0ref, diref, d2ref):
        deg = dref[...]
        dinv = jnp.where(deg > 0.0, lax.rsqrt(deg), 0.0)
        diref[...] = dinv
        d2ref[...] = dinv * dinv
        s0ref[...] = eref[...] * dinv

    return pl.pallas_call(
        body,
        grid=(NV // _R,),
        in_specs=[
            pl.BlockSpec((_R, 1), lambda i: (i, 0)),
            pl.BlockSpec((_R, D), lambda i: (i, 0)),
        ],
        out_specs=[
            pl.BlockSpec((_R, D), lambda i: (i, 0)),
            pl.BlockSpec((_R, 1), lambda i: (i, 0)),
            pl.BlockSpec((_R, 1), lambda i: (i, 0)),
        ],
        out_shape=[
            jax.ShapeDtypeStruct((NV, D), jnp.float32),
            jax.ShapeDtypeStruct((NV, 1), jnp.float32),
            jax.ShapeDtypeStruct((NV, 1), jnp.float32),
        ],
    )(deg2d, emb0)


def _scale_tc(t, dinv2):
    def body(tref, dref, oref):
        oref[...] = tref[...] * dref[...]

    return pl.pallas_call(
        body,
        grid=(NV // _R,),
        in_specs=[
            pl.BlockSpec((_R, D), lambda i: (i, 0)),
            pl.BlockSpec((_R, 1), lambda i: (i, 0)),
        ],
        out_specs=pl.BlockSpec((_R, D), lambda i: (i, 0)),
        out_shape=jax.ShapeDtypeStruct((NV, D), jnp.float32),
    )(t, dinv2)


def _final_tc(emb0, t0, t1, t2, dinv):
    def body(eref, t0ref, t1ref, t2ref, dref, oref):
        mix = 0.2 * t0ref[...] + 0.3 * t1ref[...] + 0.4 * t2ref[...]
        oref[...] = 0.1 * eref[...] + dref[...] * mix

    return pl.pallas_call(
        body,
        grid=(NV // _R,),
        in_specs=[pl.BlockSpec((_R, D), lambda i: (i, 0))] * 4
        + [pl.BlockSpec((_R, 1), lambda i: (i, 0))],
        out_specs=pl.BlockSpec((_R, D), lambda i: (i, 0)),
        out_shape=jax.ShapeDtypeStruct((NV, D), jnp.float32),
    )(emb0, t0, t1, t2, dinv)


def kernel(edge_index, users_emb, items_emb):
    row = edge_index[0].astype(jnp.int32)
    col = edge_index[1].astype(jnp.int32)
    pad = E_PAD - E
    padv = jnp.full((pad,), NV, jnp.int32)
    row1 = jnp.concatenate([row, padv])
    col1 = jnp.concatenate([col, jnp.zeros((pad,), jnp.int32)])
    padt = jnp.full((pad,), HALF, jnp.int32)
    lrow = jnp.concatenate(
        [
            jnp.where(row < HALF, row, HALF), padt,
            jnp.where(row >= HALF, row - HALF, HALF), padt,
        ]
    )
    emb0 = jnp.concatenate([users_emb, items_emb], axis=0)
    zeros2 = jnp.zeros((1024, DH), jnp.float32)
    zd = jnp.zeros((DZ,), jnp.float32)

    deg = _deg_sc(row1, zd)
    deg2d = deg[:NV].reshape(NV, 1)
    s0, dinv, dinv2 = _prep_tc(deg2d, emb0)

    def half(_, sh):
        return None, _spmm_sc(col1, lrow, sh, zeros2)

    def layer(s, _):
        s_halves = jnp.stack([s[:, :DH], s[:, DH:]])
        _, t_halves = lax.scan(half, None, s_halves)
        t = jnp.concatenate([t_halves[0], t_halves[1]], axis=1)
        return _scale_tc(t, dinv2), t

    _, ts = lax.scan(layer, s0, None, length=3)
    final = _final_tc(emb0, ts[0], ts[1], ts[2], dinv)

    uK = final[:NUM_USERS]
    iK = final[NUM_USERS:]
    return (uK, users_emb, iK, items_emb, uK, iK, uK, iK)


# speedup vs baseline: 5.7421x; 1.0268x over previous
import functools

import jax
import jax.numpy as jnp
from jax import lax
from jax.experimental import pallas as pl
from jax.experimental.pallas import tpu as pltpu
from jax.experimental.pallas import tpu_sc as plsc

NUM_USERS = 10000
NUM_ITEMS = 40000
NV = NUM_USERS + NUM_ITEMS
E = 800000
D = 64
DH = 32
NC, NS = 2, 16
NW = NC * NS
HALF = NV // NC

SCH = 4
ER = 6256
E_PAD = ER * 128
NSCH = ER // SCH
NITER = -(-NSCH // NS)
NRING = 4
ACC_ROWS = 25088
ZR = ACC_ROWS // NS
TAIL = HALF - 15 * ZR
DEG_PAD = 50048
DZ = DEG_PAD // NS
DITER = -(-NSCH // NW)

_sc_mesh = plsc.VectorSubcoreMesh(core_axis_name="c", subcore_axis_name="s")
_sc_params = pltpu.CompilerParams(use_tc_tiling_on_sc=False)


@functools.partial(
    pl.kernel,
    out_type=jax.ShapeDtypeStruct((NC, DEG_PAD), jnp.float32),
    mesh=_sc_mesh,
    compiler_params=_sc_params,
    scratch_types=[
        [[pltpu.VMEM((128,), jnp.int32) for _ in range(SCH)] for _ in range(NRING)],
        pltpu.VMEM((128,), jnp.float32),
        pltpu.VMEM((DZ,), jnp.float32),
        pltpu.VMEM_SHARED((DEG_PAD,), jnp.float32),
        [pltpu.SemaphoreType.DMA for _ in range(NRING)],
        [pltpu.SemaphoreType.DMA for _ in range(NRING)],
    ],
)
def _deg_sc(row1_h, zd_h, deg_h, rb, obuf, dbuf, dacc, isem, ssem):
    c = lax.axis_index("c")
    s = lax.axis_index("s")
    w = s * NC + c

    for m in range(8):
        obuf[pl.ds(m * 16, 16)] = jnp.full((16,), 1.0, jnp.float32)
    pltpu.sync_copy(zd_h, dbuf)
    pltpu.sync_copy(dbuf, dacc.at[pl.ds(s * DZ, DZ)])
    plsc.subcore_barrier()

    def valid(x):
        return (x >= 0) & (x * NW + w < NSCH)

    def fire_idx(x, r):
        base = (x * NW + w) * SCH * 128
        for m in range(SCH):
            pltpu.async_copy(row1_h.at[pl.ds(base + m * 128, 128)], rb[r][m], isem[r])

    def drain_idx(r):
        for m in range(SCH):
            pltpu.make_async_copy(row1_h.at[pl.ds(0, 128)], rb[r][m], isem[r]).wait()

    def fire_scat(r):
        for m in range(SCH):
            pltpu.async_copy(obuf, dacc.at[rb[r][m]], ssem[r], add=True)

    def drain_scat(r):
        for m in range(SCH):
            pltpu.make_async_copy(obuf, dacc.at[rb[r][m]], ssem[r]).wait()

    def body(g, carry):
        for q in range(NRING):
            x = g * NRING + q

            @pl.when(valid(x))
            def _(x=x, q=q):
                fire_idx(x, q)

        for q in range(NRING):
            x = g * NRING + q

            @pl.when(valid(x))
            def _(x=x, q=q):
                drain_idx(q)
                fire_scat(q)

        for q in range(NRING):
            x = g * NRING + q

            @pl.when(valid(x))
            def _(x=x, q=q):
                drain_scat(q)

        return carry

    lax.fori_loop(0, -(-DITER // NRING), body, 0)
    plsc.subcore_barrier()
    pltpu.sync_copy(dacc.at[pl.ds(s * DZ, DZ)], dbuf)
    pltpu.sync_copy(dbuf, deg_h.at[c, pl.ds(s * DZ, DZ)])


@functools.partial(
    pl.kernel,
    out_type=jax.ShapeDtypeStruct((NV, DH), jnp.float32),
    mesh=_sc_mesh,
    compiler_params=_sc_params,
    scratch_types=[
        [[pltpu.VMEM((128,), jnp.int32) for _ in range(SCH)] for _ in range(NRING)],
        [[pltpu.VMEM((128,), jnp.int32) for _ in range(SCH)] for _ in range(NRING)],
        [pltpu.VMEM((SCH * 128, DH), jnp.float32) for _ in range(NRING)],
        pltpu.VMEM_SHARED((ACC_ROWS, DH), jnp.float32),
        [pltpu.SemaphoreType.DMA for _ in range(NRING)],
        [pltpu.SemaphoreType.DMA for _ in range(NRING)],
        [pltpu.SemaphoreType.DMA for _ in range(NRING)],
    ],
)
def _spmm_sc(col1_h, lrow_h, s_h, zeros_h, out_h, cb, lb, gb, acc, isem, gsem, ssem):
    c = lax.axis_index("c")
    s = lax.axis_index("s")
    lbase = c * E_PAD

    pltpu.sync_copy(zeros_h, gb[0])
    for q in range(3):
        pltpu.sync_copy(gb[0], acc.at[pl.ds(s * ZR + q * 512, 512)])
    pltpu.sync_copy(gb[0].at[pl.ds(0, 32)], acc.at[pl.ds(s * ZR + 1536, 32)])
    plsc.subcore_barrier()

    def valid(x):
        return (x >= 0) & (x * NS + s < NSCH)

    def fire_idx(x, r):
        base = (x * NS + s) * SCH * 128
        for m in range(SCH):
            pltpu.async_copy(col1_h.at[pl.ds(base + m * 128, 128)], cb[r][m], isem[r])
            pltpu.async_copy(
                lrow_h.at[pl.ds(lbase + base + m * 128, 128)], lb[r][m], isem[r]
            )

    def drain_idx(r):
        for m in range(SCH):
            pltpu.make_async_copy(col1_h.at[pl.ds(0, 128)], cb[r][m], isem[r]).wait()
            pltpu.make_async_copy(col1_h.at[pl.ds(0, 128)], lb[r][m], isem[r]).wait()

    def fire_gath(r):
        for m in range(SCH):
            pltpu.async_copy(
                s_h.at[cb[r][m]], gb[r].at[pl.ds(m * 128, 128)], gsem[r]
            )

    def drain_gath(r):
        for m in range(SCH):
            pltpu.make_async_copy(
                s_h.at[cb[r][m]], gb[r].at[pl.ds(m * 128, 128)], gsem[r]
            ).wait()

    def fire_scat(r):
        for m in range(SCH):
            pltpu.async_copy(
                gb[r].at[pl.ds(m * 128, 128)], acc.at[lb[r][m]], ssem[r], add=True
            )

    def drain_scat(r):
        for m in range(SCH):
            pltpu.make_async_copy(
                gb[r].at[pl.ds(m * 128, 128)], acc.at[lb[r][m]], ssem[r]
            ).wait()

    def body(g, carry):
        for q in range(NRING):
            x = g * NRING + q

            @pl.when(valid(x))
            def _(x=x, q=q):
                fire_idx(x, q)

        for q in range(NRING):
            x = g * NRING + q

            @pl.when(valid(x))
            def _(x=x, q=q):
                drain_idx(q)
                fire_gath(q)

        for q in range(NRING):
            x = g * NRING + q

            @pl.when(valid(x))
            def _(x=x, q=q):
                drain_gath(q)
                fire_scat(q)

        for q in range(NRING):
            x = g * NRING + q

            @pl.when(valid(x))
            def _(x=x, q=q):
                drain_scat(q)

        return carry

    lax.fori_loop(0, -(-NITER // NRING), body, 0)
    plsc.subcore_barrier()

    def copy_out(aoff, ooff, n):
        pltpu.sync_copy(acc.at[pl.ds(aoff, n)], gb[0].at[pl.ds(0, n)])
        pltpu.sync_copy(gb[0].at[pl.ds(0, n)], out_h.at[pl.ds(ooff, n)])

    @pl.when(s < NS - 1)
    def _():
        for q in range(3):
            copy_out(s * ZR + q * 512, c * HALF + s * ZR + q * 512, 512)
        copy_out(s * ZR + 1536, c * HALF + s * ZR + 1536, 32)

    @pl.when(s == NS - 1)
    def _():
        for q in range(2):
            copy_out((NS - 1) * ZR + q * 512, c * HALF + (NS - 1) * ZR + q * 512, 512)
        copy_out((NS - 1) * ZR + 1024, c * HALF + (NS - 1) * ZR + 1024, TAIL - 1024)


_R = 5000


def _prep_tc(degp, emb0):
    def body(dref, eref, s0ref, diref, d2ref):
        deg = dref[0] + dref[1]
        dinv = jnp.where(deg > 0.0, lax.rsqrt(deg), 0.0)
        diref[...] = dinv
        d2ref[...] = dinv * dinv
        s0ref[0] = eref[:, :DH] * dinv
        s0ref[1] = eref[:, DH:] * dinv

    return pl.pallas_call(
        body,
        grid=(NV // _R,),
        in_specs=[
            pl.BlockSpec((2, _R, 1), lambda i: (0, i, 0)),
            pl.BlockSpec((_R, D), lambda i: (i, 0)),
        ],
        out_specs=[
            pl.BlockSpec((2, _R, DH), lambda i: (0, i, 0)),
            pl.BlockSpec((_R, 1), lambda i: (i, 0)),
            pl.BlockSpec((_R, 1), lambda i: (i, 0)),
        ],
        out_shape=[
            jax.ShapeDtypeStruct((2, NV, DH), jnp.float32),
            jax.ShapeDtypeStruct((NV, 1), jnp.float32),
            jax.ShapeDtypeStruct((NV, 1), jnp.float32),
        ],
    )(degp, emb0)


def _scale_tc(t, dinv2):
    def body(tref, dref, oref):
        oref[...] = tref[...] * dref[...]

    return pl.pallas_call(
        body,
        grid=(2, NV // _R),
        in_specs=[
            pl.BlockSpec((1, _R, DH), lambda h, i: (h, i, 0)),
            pl.BlockSpec((_R, 1), lambda h, i: (i, 0)),
        ],
        out_specs=pl.BlockSpec((1, _R, DH), lambda h, i: (h, i, 0)),
        out_shape=jax.ShapeDtypeStruct((2, NV, DH), jnp.float32),
    )(t, dinv2)


def _final_tc(emb0, ts, dinv):
    def body(eref, t0ref, t1ref, t2ref, dref, oref):
        d = dref[...]
        mix_a = d * (0.2 * t0ref[0, 0] + 0.3 * t1ref[0, 0] + 0.4 * t2ref[0, 0])
        mix_b = d * (0.2 * t0ref[0, 1] + 0.3 * t1ref[0, 1] + 0.4 * t2ref[0, 1])
        oref[...] = 0.1 * eref[...] + jnp.concatenate([mix_a, mix_b], axis=1)

    tspec = lambda l: pl.BlockSpec((1, 2, _R, DH), lambda i, l=l: (l, 0, i, 0))
    return pl.pallas_call(
        body,
        grid=(NV // _R,),
        in_specs=[
            pl.BlockSpec((_R, D), lambda i: (i, 0)),
            tspec(0),
            tspec(1),
            tspec(2),
            pl.BlockSpec((_R, 1), lambda i: (i, 0)),
        ],
        out_specs=pl.BlockSpec((_R, D), lambda i: (i, 0)),
        out_shape=jax.ShapeDtypeStruct((NV, D), jnp.float32),
    )(emb0, ts, ts, ts, dinv)


def kernel(edge_index, users_emb, items_emb):
    row = edge_index[0].astype(jnp.int32)
    col = edge_index[1].astype(jnp.int32)
    pad = E_PAD - E
    row1 = jnp.concatenate([row, jnp.full((pad,), NV, jnp.int32)])
    col1 = jnp.concatenate([col, jnp.zeros((pad,), jnp.int32)])
    padt = jnp.full((pad,), HALF, jnp.int32)
    lrow = jnp.concatenate(
        [
            jnp.where(row < HALF, row, HALF), padt,
            jnp.where(row >= HALF, row - HALF, HALF), padt,
        ]
    )
    emb0 = jnp.concatenate([users_emb, items_emb], axis=0)
    zeros2 = jnp.zeros((SCH * 128, DH), jnp.float32)
    zd = jnp.zeros((DZ,), jnp.float32)

    degp = _deg_sc(row1, zd)
    degp = degp[:, :NV].reshape(NC, NV, 1)
    s0, dinv, dinv2 = _prep_tc(degp, emb0)

    def half(_, sh):
        return None, _spmm_sc(col1, lrow, sh, zeros2)

    def layer(s, _):
        _, t_halves = lax.scan(half, None, s)
        return _scale_tc(t_halves, dinv2), t_halves

    _, ts = lax.scan(layer, s0, None, length=3)
    final = _final_tc(emb0, ts, dinv)

    uK = final[:NUM_USERS]
    iK = final[NUM_USERS:]
    return (uK, users_emb, iK, items_emb, uK, iK, uK, iK)


# speedup vs baseline: 10.8824x; 1.8952x over previous
import functools

import jax
import jax.numpy as jnp
from jax import lax
from jax.experimental import pallas as pl
from jax.experimental.pallas import tpu as pltpu
from jax.experimental.pallas import tpu_sc as plsc

NUM_USERS = 10000
NUM_ITEMS = 40000
NV = NUM_USERS + NUM_ITEMS
E = 800000
D = 64
DH = 32
NC, NS = 2, 16
NW = NC * NS
HALF = NV // NC

SCH = 4
ER = 6256
E_PAD = ER * 128
NSCH = ER // SCH
NITER = -(-NSCH // NS)
NRING = 4
ACC_ROWS = 25088
ZR = ACC_ROWS // NS
TAIL = HALF - 15 * ZR
DEG_PAD = 50048
DZ = DEG_PAD // NS
DITER = -(-NSCH // NW)

_sc_mesh = plsc.VectorSubcoreMesh(core_axis_name="c", subcore_axis_name="s")
_sc_params = pltpu.CompilerParams(use_tc_tiling_on_sc=False)


@functools.partial(
    pl.kernel,
    out_type=jax.ShapeDtypeStruct((NC, DEG_PAD), jnp.float32),
    mesh=_sc_mesh,
    compiler_params=_sc_params,
    scratch_types=[
        [[pltpu.VMEM((128,), jnp.int32) for _ in range(SCH)] for _ in range(NRING)],
        pltpu.VMEM((128,), jnp.float32),
        pltpu.VMEM((DZ,), jnp.float32),
        pltpu.VMEM_SHARED((DEG_PAD,), jnp.float32),
        [pltpu.SemaphoreType.DMA for _ in range(NRING)],
        [pltpu.SemaphoreType.DMA for _ in range(NRING)],
    ],
)
def _deg_sc(row1_h, zd_h, deg_h, rb, obuf, dbuf, dacc, isem, ssem):
    c = lax.axis_index("c")
    s = lax.axis_index("s")
    w = s * NC + c

    for m in range(8):
        obuf[pl.ds(m * 16, 16)] = jnp.full((16,), 1.0, jnp.float32)
    pltpu.sync_copy(zd_h, dbuf)
    pltpu.sync_copy(dbuf, dacc.at[pl.ds(s * DZ, DZ)])
    plsc.subcore_barrier()

    def valid(x):
        return (x >= 0) & (x * NW + w < NSCH)

    def fire_idx(x, r):
        base = (x * NW + w) * SCH * 128
        for m in range(SCH):
            pltpu.async_copy(row1_h.at[pl.ds(base + m * 128, 128)], rb[r][m], isem[r])

    def drain_idx(r):
        for m in range(SCH):
            pltpu.make_async_copy(row1_h.at[pl.ds(0, 128)], rb[r][m], isem[r]).wait()

    def fire_scat(r):
        for m in range(SCH):
            pltpu.async_copy(obuf, dacc.at[rb[r][m]], ssem[r], add=True)

    def drain_scat(r):
        for m in range(SCH):
            pltpu.make_async_copy(obuf, dacc.at[rb[r][m]], ssem[r]).wait()

    def body(g, carry):
        for q in range(NRING):
            x = g * NRING + q

            @pl.when(valid(x))
            def _(x=x, q=q):
                fire_idx(x, q)

        for q in range(NRING):
            x = g * NRING + q

            @pl.when(valid(x))
            def _(x=x, q=q):
                drain_idx(q)
                fire_scat(q)

        for q in range(NRING):
            x = g * NRING + q

            @pl.when(valid(x))
            def _(x=x, q=q):
                drain_scat(q)

        return carry

    lax.fori_loop(0, -(-DITER // NRING), body, 0)
    plsc.subcore_barrier()
    pltpu.sync_copy(dacc.at[pl.ds(s * DZ, DZ)], dbuf)
    pltpu.sync_copy(dbuf, deg_h.at[c, pl.ds(s * DZ, DZ)])


@functools.partial(
    pl.kernel,
    out_type=jax.ShapeDtypeStruct((NV, DH), jnp.float32),
    mesh=_sc_mesh,
    compiler_params=_sc_params,
    scratch_types=[
        [[pltpu.VMEM((128,), jnp.int32) for _ in range(SCH)] for _ in range(NRING)],
        [[pltpu.VMEM((128,), jnp.int32) for _ in range(SCH)] for _ in range(NRING)],
        [pltpu.VMEM((SCH * 128, DH), jnp.float32) for _ in range(NRING)],
        pltpu.VMEM_SHARED((ACC_ROWS, DH), jnp.float32),
        [pltpu.SemaphoreType.DMA for _ in range(NRING)],
        [pltpu.SemaphoreType.DMA for _ in range(NRING)],
        [pltpu.SemaphoreType.DMA for _ in range(NRING)],
    ],
)
def _spmm_sc(col1_h, lrow_h, s_h, zeros_h, out_h, cb, lb, gb, acc, isem, gsem, ssem):
    c = lax.axis_index("c")
    s = lax.axis_index("s")
    lbase = c * E_PAD

    pltpu.sync_copy(zeros_h, gb[0])
    for q in range(3):
        pltpu.sync_copy(gb[0], acc.at[pl.ds(s * ZR + q * 512, 512)])
    pltpu.sync_copy(gb[0].at[pl.ds(0, 32)], acc.at[pl.ds(s * ZR + 1536, 32)])
    plsc.subcore_barrier()

    def valid(x):
        return (x >= 0) & (x * NS + s < NSCH)

    def fire_idx(x, r):
        base = (x * NS + s) * SCH * 128
        for m in range(SCH):
            pltpu.async_copy(col1_h.at[pl.ds(base + m * 128, 128)], cb[r][m], isem[r])
            pltpu.async_copy(
                lrow_h.at[pl.ds(lbase + base + m * 128, 128)], lb[r][m], isem[r]
            )

    def drain_idx(r):
        for m in range(SCH):
            pltpu.make_async_copy(col1_h.at[pl.ds(0, 128)], cb[r][m], isem[r]).wait()
            pltpu.make_async_copy(col1_h.at[pl.ds(0, 128)], lb[r][m], isem[r]).wait()

    def fire_gath(r):
        for m in range(SCH):
            pltpu.async_copy(
                s_h.at[cb[r][m]], gb[r].at[pl.ds(m * 128, 128)], gsem[r]
            )

    def drain_gath(r):
        for m in range(SCH):
            pltpu.make_async_copy(
                s_h.at[cb[r][m]], gb[r].at[pl.ds(m * 128, 128)], gsem[r]
            ).wait()

    def fire_scat(r):
        for m in range(SCH):
            pltpu.async_copy(
                gb[r].at[pl.ds(m * 128, 128)], acc.at[lb[r][m]], ssem[r], add=True
            )

    def drain_scat(r):
        for m in range(SCH):
            pltpu.make_async_copy(
                gb[r].at[pl.ds(m * 128, 128)], acc.at[lb[r][m]], ssem[r]
            ).wait()

    def body(g, carry):
        for q in range(NRING):
            x = g * NRING + q

            @pl.when(valid(x))
            def _(x=x, q=q):
                fire_idx(x, q)

        for q in range(NRING):
            x = g * NRING + q

            @pl.when(valid(x))
            def _(x=x, q=q):
                drain_idx(q)
                fire_gath(q)

        for q in range(NRING):
            x = g * NRING + q

            @pl.when(valid(x))
            def _(x=x, q=q):
                drain_gath(q)
                fire_scat(q)

        for q in range(NRING):
            x = g * NRING + q

            @pl.when(valid(x))
            def _(x=x, q=q):
                drain_scat(q)

        return carry

    lax.fori_loop(0, -(-NITER // NRING), body, 0)
    plsc.subcore_barrier()

    def copy_out(aoff, ooff, n):
        pltpu.sync_copy(acc.at[pl.ds(aoff, n)], gb[0].at[pl.ds(0, n)])
        pltpu.sync_copy(gb[0].at[pl.ds(0, n)], out_h.at[pl.ds(ooff, n)])

    @pl.when(s < NS - 1)
    def _():
        for q in range(3):
            copy_out(s * ZR + q * 512, c * HALF + s * ZR + q * 512, 512)
        copy_out(s * ZR + 1536, c * HALF + s * ZR + 1536, 32)

    @pl.when(s == NS - 1)
    def _():
        for q in range(2):
            copy_out((NS - 1) * ZR + q * 512, c * HALF + (NS - 1) * ZR + q * 512, 512)
        copy_out((NS - 1) * ZR + 1024, c * HALF + (NS - 1) * ZR + 1024, TAIL - 1024)


_R = 5000


def _prep_tc(degp, emb0):
    def body(dref, eref, s0ref, diref, d2ref):
        deg = dref[0] + dref[1]
        dinv = jnp.where(deg > 0.0, lax.rsqrt(deg), 0.0)
        diref[...] = dinv
        d2ref[...] = dinv * dinv
        s0ref[0] = eref[:, :DH] * dinv
        s0ref[1] = eref[:, DH:] * dinv

    return pl.pallas_call(
        body,
        grid=(NV // _R,),
        in_specs=[
            pl.BlockSpec((2, _R, 1), lambda i: (0, i, 0)),
            pl.BlockSpec((_R, D), lambda i: (i, 0)),
        ],
        out_specs=[
            pl.BlockSpec((2, _R, DH), lambda i: (0, i, 0)),
            pl.BlockSpec((_R, 1), lambda i: (i, 0)),
            pl.BlockSpec((_R, 1), lambda i: (i, 0)),
        ],
        out_shape=[
            jax.ShapeDtypeStruct((2, NV, DH), jnp.float32),
            jax.ShapeDtypeStruct((NV, 1), jnp.float32),
            jax.ShapeDtypeStruct((NV, 1), jnp.float32),
        ],
    )(degp, emb0)


def _scale_tc(t, dinv2):
    def body(tref, dref, oref):
        oref[...] = tref[...] * dref[...]

    return pl.pallas_call(
        body,
        grid=(2, NV // _R),
        in_specs=[
            pl.BlockSpec((1, _R, DH), lambda h, i: (h, i, 0)),
            pl.BlockSpec((_R, 1), lambda h, i: (i, 0)),
        ],
        out_specs=pl.BlockSpec((1, _R, DH), lambda h, i: (h, i, 0)),
        out_shape=jax.ShapeDtypeStruct((2, NV, DH), jnp.float32),
    )(t, dinv2)


def _final_tc(emb0, ts, dinv):
    def body(eref, t0ref, t1ref, t2ref, dref, oref):
        d = dref[...]
        mix_a = d * (0.2 * t0ref[0, 0] + 0.3 * t1ref[0, 0] + 0.4 * t2ref[0, 0])
        mix_b = d * (0.2 * t0ref[0, 1] + 0.3 * t1ref[0, 1] + 0.4 * t2ref[0, 1])
        oref[...] = 0.1 * eref[...] + jnp.concatenate([mix_a, mix_b], axis=1)

    tspec = lambda l: pl.BlockSpec((1, 2, _R, DH), lambda i, l=l: (l, 0, i, 0))
    return pl.pallas_call(
        body,
        grid=(NV // _R,),
        in_specs=[
            pl.BlockSpec((_R, D), lambda i: (i, 0)),
            tspec(0),
            tspec(1),
            tspec(2),
            pl.BlockSpec((_R, 1), lambda i: (i, 0)),
        ],
        out_specs=pl.BlockSpec((_R, D), lambda i: (i, 0)),
        out_shape=jax.ShapeDtypeStruct((NV, D), jnp.float32),
    )(emb0, ts, ts, ts, dinv)


def kernel(edge_index, users_emb, items_emb):
    row = edge_index[0].astype(jnp.int32)
    col = edge_index[1].astype(jnp.int32)
    pad = E_PAD - E
    row1 = jnp.concatenate([row, jnp.full((pad,), NV, jnp.int32)])
    col1 = jnp.concatenate([col, jnp.zeros((pad,), jnp.int32)])
    trash = HALF + jnp.arange(E, dtype=jnp.int32) % (ACC_ROWS - HALF)
    padt = HALF + jnp.arange(pad, dtype=jnp.int32) % (ACC_ROWS - HALF)
    lrow = jnp.concatenate(
        [
            jnp.where(row < HALF, row, trash), padt,
            jnp.where(row >= HALF, row - HALF, trash), padt,
        ]
    )
    emb0 = jnp.concatenate([users_emb, items_emb], axis=0)
    zeros2 = jnp.zeros((SCH * 128, DH), jnp.float32)
    zd = jnp.zeros((DZ,), jnp.float32)

    degp = _deg_sc(row1, zd)
    degp = degp[:, :NV].reshape(NC, NV, 1)
    s0, dinv, dinv2 = _prep_tc(degp, emb0)

    def half(_, sh):
        return None, _spmm_sc(col1, lrow, sh, zeros2)

    def layer(s, _):
        _, t_halves = lax.scan(half, None, s)
        return _scale_tc(t_halves, dinv2), t_halves

    _, ts = lax.scan(layer, s0, None, length=3)
    final = _final_tc(emb0, ts, dinv)

    uK = final[:NUM_USERS]
    iK = final[NUM_USERS:]
    return (uK, users_emb, iK, items_emb, uK, iK, uK, iK)


# speedup vs baseline: 11.7620x; 1.0808x over previous
import functools

import jax
import jax.numpy as jnp
from jax import lax
from jax.experimental import pallas as pl
from jax.experimental.pallas import tpu as pltpu
from jax.experimental.pallas import tpu_sc as plsc

NUM_USERS = 10000
NUM_ITEMS = 40000
NV = NUM_USERS + NUM_ITEMS
E = 800000
D = 64
DH = 32
NC, NS = 2, 16
NW = NC * NS
HALF = NV // NC

SCH = 4
ER = 6256
E_PAD = ER * 128
NSCH = ER // SCH
NITER = -(-NSCH // NS)
NRING = 4
ACC_ROWS = 25088
ZR = ACC_ROWS // NS
TAIL = HALF - 15 * ZR
DEG_PAD = 50048
DZ = DEG_PAD // NS
DITER = -(-NSCH // NW)

_sc_mesh = plsc.VectorSubcoreMesh(core_axis_name="c", subcore_axis_name="s")
_sc_params = pltpu.CompilerParams(use_tc_tiling_on_sc=False)


@functools.partial(
    pl.kernel,
    out_type=jax.ShapeDtypeStruct((NC, DEG_PAD), jnp.float32),
    mesh=_sc_mesh,
    compiler_params=_sc_params,
    scratch_types=[
        [[pltpu.VMEM((128,), jnp.int32) for _ in range(SCH)] for _ in range(NRING)],
        pltpu.VMEM((128,), jnp.float32),
        pltpu.VMEM((DZ,), jnp.float32),
        pltpu.VMEM_SHARED((DEG_PAD,), jnp.float32),
        [pltpu.SemaphoreType.DMA for _ in range(NRING)],
        [pltpu.SemaphoreType.DMA for _ in range(NRING)],
    ],
)
def _deg_sc(row1_h, zd_h, deg_h, rb, obuf, dbuf, dacc, isem, ssem):
    c = lax.axis_index("c")
    s = lax.axis_index("s")
    w = s * NC + c

    for m in range(8):
        obuf[pl.ds(m * 16, 16)] = jnp.full((16,), 1.0, jnp.float32)
    pltpu.sync_copy(zd_h, dbuf)
    pltpu.sync_copy(dbuf, dacc.at[pl.ds(s * DZ, DZ)])
    plsc.subcore_barrier()

    def valid(x):
        return (x >= 0) & (x * NW + w < NSCH)

    def fire_idx(x, r):
        base = (x * NW + w) * SCH * 128
        for m in range(SCH):
            pltpu.async_copy(row1_h.at[pl.ds(base + m * 128, 128)], rb[r][m], isem[r])

    def drain_idx(r):
        for m in range(SCH):
            pltpu.make_async_copy(row1_h.at[pl.ds(0, 128)], rb[r][m], isem[r]).wait()

    def fire_scat(r):
        for m in range(SCH):
            pltpu.async_copy(obuf, dacc.at[rb[r][m]], ssem[r], add=True)

    def drain_scat(r):
        for m in range(SCH):
            pltpu.make_async_copy(obuf, dacc.at[rb[r][m]], ssem[r]).wait()

    def body(g, carry):
        for q in range(NRING):
            x = g * NRING + q

            @pl.when(valid(x))
            def _(x=x, q=q):
                fire_idx(x, q)

        for q in range(NRING):
            x = g * NRING + q

            @pl.when(valid(x))
            def _(x=x, q=q):
                drain_idx(q)
                fire_scat(q)

        for q in range(NRING):
            x = g * NRING + q

            @pl.when(valid(x))
            def _(x=x, q=q):
                drain_scat(q)

        return carry

    lax.fori_loop(0, -(-DITER // NRING), body, 0)
    plsc.subcore_barrier()
    pltpu.sync_copy(dacc.at[pl.ds(s * DZ, DZ)], dbuf)
    pltpu.sync_copy(dbuf, deg_h.at[c, pl.ds(s * DZ, DZ)])


@functools.partial(
    pl.kernel,
    out_type=[
        jax.ShapeDtypeStruct((NV, DH), jnp.float32),
        jax.ShapeDtypeStruct((NV, DH), jnp.float32),
    ],
    mesh=_sc_mesh,
    compiler_params=_sc_params,
    scratch_types=[
        [[pltpu.VMEM((128,), jnp.int32) for _ in range(SCH)] for _ in range(NRING)],
        [[pltpu.VMEM((128,), jnp.int32) for _ in range(SCH)] for _ in range(NRING)],
        [pltpu.VMEM((SCH * 128, DH), jnp.float32) for _ in range(NRING)],
        pltpu.VMEM_SHARED((ACC_ROWS, DH), jnp.float32),
        [pltpu.SemaphoreType.DMA for _ in range(NRING)],
        [pltpu.SemaphoreType.DMA for _ in range(NRING)],
        [pltpu.SemaphoreType.DMA for _ in range(NRING)],
    ],
)
def _spmm_sc(col1_h, lrow_h, sa_h, sb_h, zeros_h, ta_h, tb_h,
             cb, lb, gb, acc, isem, gsem, ssem):
    c = lax.axis_index("c")
    s = lax.axis_index("s")
    lbase = c * E_PAD

    def valid(x):
        return (x >= 0) & (x * NS + s < NSCH)

    def fire_idx(x, r):
        base = (x * NS + s) * SCH * 128
        for m in range(SCH):
            pltpu.async_copy(col1_h.at[pl.ds(base + m * 128, 128)], cb[r][m], isem[r])
            pltpu.async_copy(
                lrow_h.at[pl.ds(lbase + base + m * 128, 128)], lb[r][m], isem[r]
            )

    def drain_idx(r):
        for m in range(SCH):
            pltpu.make_async_copy(col1_h.at[pl.ds(0, 128)], cb[r][m], isem[r]).wait()
            pltpu.make_async_copy(col1_h.at[pl.ds(0, 128)], lb[r][m], isem[r]).wait()

    def fire_gath(s_h, r):
        for m in range(SCH):
            pltpu.async_copy(
                s_h.at[cb[r][m]], gb[r].at[pl.ds(m * 128, 128)], gsem[r]
            )

    def drain_gath(s_h, r):
        for m in range(SCH):
            pltpu.make_async_copy(
                s_h.at[cb[r][m]], gb[r].at[pl.ds(m * 128, 128)], gsem[r]
            ).wait()

    def fire_scat(r):
        for m in range(SCH):
            pltpu.async_copy(
                gb[r].at[pl.ds(m * 128, 128)], acc.at[lb[r][m]], ssem[r], add=True
            )

    def drain_scat(r):
        for m in range(SCH):
            pltpu.make_async_copy(
                gb[r].at[pl.ds(m * 128, 128)], acc.at[lb[r][m]], ssem[r]
            ).wait()

    def copy_out(out_h, aoff, ooff, n):
        pltpu.sync_copy(acc.at[pl.ds(aoff, n)], gb[0].at[pl.ds(0, n)])
        pltpu.sync_copy(gb[0].at[pl.ds(0, n)], out_h.at[pl.ds(ooff, n)])

    for s_h, out_h in ((sa_h, ta_h), (sb_h, tb_h)):
        pltpu.sync_copy(zeros_h, gb[0])
        for q in range(3):
            pltpu.sync_copy(gb[0], acc.at[pl.ds(s * ZR + q * 512, 512)])
        pltpu.sync_copy(gb[0].at[pl.ds(0, 32)], acc.at[pl.ds(s * ZR + 1536, 32)])
        plsc.subcore_barrier()

        def body(g, carry, s_h=s_h):
            for q in range(NRING):
                x = g * NRING + q

                @pl.when(valid(x))
                def _(x=x, q=q):
                    fire_idx(x, q)

            for q in range(NRING):
                x = g * NRING + q

                @pl.when(valid(x))
                def _(x=x, q=q):
                    drain_idx(q)
                    fire_gath(s_h, q)

            for q in range(NRING):
                x = g * NRING + q

                @pl.when(valid(x))
                def _(x=x, q=q):
                    drain_gath(s_h, q)
                    fire_scat(q)

            for q in range(NRING):
                x = g * NRING + q

                @pl.when(valid(x))
                def _(x=x, q=q):
                    drain_scat(q)

            return carry

        lax.fori_loop(0, -(-NITER // NRING), body, 0)
        plsc.subcore_barrier()

        @pl.when(s < NS - 1)
        def _(out_h=out_h):
            for q in range(3):
                copy_out(out_h, s * ZR + q * 512, c * HALF + s * ZR + q * 512, 512)
            copy_out(out_h, s * ZR + 1536, c * HALF + s * ZR + 1536, 32)

        @pl.when(s == NS - 1)
        def _(out_h=out_h):
            for q in range(2):
                copy_out(out_h, (NS - 1) * ZR + q * 512,
                         c * HALF + (NS - 1) * ZR + q * 512, 512)
            copy_out(out_h, (NS - 1) * ZR + 1024,
                     c * HALF + (NS - 1) * ZR + 1024, TAIL - 1024)


_R = 5000


def _prep_tc(degp, emb0):
    def body(dref, eref, saref, sbref, diref, d2ref):
        deg = dref[0] + dref[1]
        dinv = jnp.where(deg > 0.0, lax.rsqrt(deg), 0.0)
        diref[...] = dinv
        d2ref[...] = dinv * dinv
        saref[...] = eref[:, :DH] * dinv
        sbref[...] = eref[:, DH:] * dinv

    return pl.pallas_call(
        body,
        grid=(NV // _R,),
        in_specs=[
            pl.BlockSpec((2, _R, 1), lambda i: (0, i, 0)),
            pl.BlockSpec((_R, D), lambda i: (i, 0)),
        ],
        out_specs=[
            pl.BlockSpec((_R, DH), lambda i: (i, 0)),
            pl.BlockSpec((_R, DH), lambda i: (i, 0)),
            pl.BlockSpec((_R, 1), lambda i: (i, 0)),
            pl.BlockSpec((_R, 1), lambda i: (i, 0)),
        ],
        out_shape=[
            jax.ShapeDtypeStruct((NV, DH), jnp.float32),
            jax.ShapeDtypeStruct((NV, DH), jnp.float32),
            jax.ShapeDtypeStruct((NV, 1), jnp.float32),
            jax.ShapeDtypeStruct((NV, 1), jnp.float32),
        ],
    )(degp, emb0)


def _scale_tc(ta, tb, dinv2):
    def body(taref, tbref, dref, oaref, obref):
        d = dref[...]
        oaref[...] = taref[...] * d
        obref[...] = tbref[...] * d

    return pl.pallas_call(
        body,
        grid=(NV // _R,),
        in_specs=[
            pl.BlockSpec((_R, DH), lambda i: (i, 0)),
            pl.BlockSpec((_R, DH), lambda i: (i, 0)),
            pl.BlockSpec((_R, 1), lambda i: (i, 0)),
        ],
        out_specs=[
            pl.BlockSpec((_R, DH), lambda i: (i, 0)),
            pl.BlockSpec((_R, DH), lambda i: (i, 0)),
        ],
        out_shape=[
            jax.ShapeDtypeStruct((NV, DH), jnp.float32),
            jax.ShapeDtypeStruct((NV, DH), jnp.float32),
        ],
    )(ta, tb, dinv2)


def _final_tc(emb0, tsa, tsb, dinv):
    def body(eref, a0, a1, a2, b0, b1, b2, dref, oref):
        d = dref[...]
        mix_a = d * (0.2 * a0[0] + 0.3 * a1[0] + 0.4 * a2[0])
        mix_b = d * (0.2 * b0[0] + 0.3 * b1[0] + 0.4 * b2[0])
        oref[...] = 0.1 * eref[...] + jnp.concatenate([mix_a, mix_b], axis=1)

    tspec = lambda l: pl.BlockSpec((1, _R, DH), lambda i, l=l: (l, i, 0))
    return pl.pallas_call(
        body,
        grid=(NV // _R,),
        in_specs=[pl.BlockSpec((_R, D), lambda i: (i, 0))]
        + [tspec(l) for l in range(3)] * 2
        + [pl.BlockSpec((_R, 1), lambda i: (i, 0))],
        out_specs=pl.BlockSpec((_R, D), lambda i: (i, 0)),
        out_shape=jax.ShapeDtypeStruct((NV, D), jnp.float32),
    )(emb0, tsa, tsa, tsa, tsb, tsb, tsb, dinv)


def kernel(edge_index, users_emb, items_emb):
    row = edge_index[0].astype(jnp.int32)
    col = edge_index[1].astype(jnp.int32)
    pad = E_PAD - E
    row1 = jnp.concatenate([row, jnp.full((pad,), NV, jnp.int32)])
    col1 = jnp.concatenate([col, jnp.zeros((pad,), jnp.int32)])
    trash = HALF + jnp.arange(E, dtype=jnp.int32) % (ACC_ROWS - HALF)
    padt = HALF + jnp.arange(pad, dtype=jnp.int32) % (ACC_ROWS - HALF)
    lrow = jnp.concatenate(
        [
            jnp.where(row < HALF, row, trash), padt,
            jnp.where(row >= HALF, row - HALF, trash), padt,
        ]
    )
    emb0 = jnp.concatenate([users_emb, items_emb], axis=0)
    zeros2 = jnp.zeros((SCH * 128, DH), jnp.float32)
    zd = jnp.zeros((DZ,), jnp.float32)

    degp = _deg_sc(row1, zd)
    degp = degp[:, :NV].reshape(NC, NV, 1)
    s0a, s0b, dinv, dinv2 = _prep_tc(degp, emb0)

    def layer(s, _):
        sa, sb = s
        ta, tb = _spmm_sc(col1, lrow, sa, sb, zeros2)
        return _scale_tc(ta, tb, dinv2), (ta, tb)

    _, (tsa, tsb) = lax.scan(layer, (s0a, s0b), None, length=3)
    final = _final_tc(emb0, tsa, tsb, dinv)

    uK = final[:NUM_USERS]
    iK = final[NUM_USERS:]
    return (uK, users_emb, iK, items_emb, uK, iK, uK, iK)


# speedup vs baseline: 12.2508x; 1.0416x over previous
import functools

import jax
import jax.numpy as jnp
from jax import lax
from jax.experimental import pallas as pl
from jax.experimental.pallas import tpu as pltpu
from jax.experimental.pallas import tpu_sc as plsc

NUM_USERS = 10000
NUM_ITEMS = 40000
NV = NUM_USERS + NUM_ITEMS
E = 800000
D = 64
DH = 32
NC, NS = 2, 16
NW = NC * NS
HALF = NV // NC

SCH = 2
ER = 6256
E_PAD = ER * 128
NSCH = ER // SCH
NITER = -(-NSCH // NS)
NRING = 8
ACC_ROWS = 25088
ZR = ACC_ROWS // NS
TAIL = HALF - 15 * ZR
DEG_PAD = 50048
DZ = DEG_PAD // NS
DITER = -(-NSCH // NW)

_sc_mesh = plsc.VectorSubcoreMesh(core_axis_name="c", subcore_axis_name="s")
_sc_params = pltpu.CompilerParams(use_tc_tiling_on_sc=False)


@functools.partial(
    pl.kernel,
    out_type=jax.ShapeDtypeStruct((NC, DEG_PAD), jnp.float32),
    mesh=_sc_mesh,
    compiler_params=_sc_params,
    scratch_types=[
        [[pltpu.VMEM((128,), jnp.int32) for _ in range(SCH)] for _ in range(NRING)],
        pltpu.VMEM((128,), jnp.float32),
        pltpu.VMEM((DZ,), jnp.float32),
        pltpu.VMEM_SHARED((DEG_PAD,), jnp.float32),
        [pltpu.SemaphoreType.DMA for _ in range(NRING)],
        [pltpu.SemaphoreType.DMA for _ in range(NRING)],
    ],
)
def _deg_sc(row1_h, zd_h, deg_h, rb, obuf, dbuf, dacc, isem, ssem):
    c = lax.axis_index("c")
    s = lax.axis_index("s")
    w = s * NC + c

    for m in range(8):
        obuf[pl.ds(m * 16, 16)] = jnp.full((16,), 1.0, jnp.float32)
    pltpu.sync_copy(zd_h, dbuf)
    pltpu.sync_copy(dbuf, dacc.at[pl.ds(s * DZ, DZ)])
    plsc.subcore_barrier()

    def valid(x):
        return (x >= 0) & (x * NW + w < NSCH)

    def fire_idx(x, r):
        base = (x * NW + w) * SCH * 128
        for m in range(SCH):
            pltpu.async_copy(row1_h.at[pl.ds(base + m * 128, 128)], rb[r][m], isem[r])

    def drain_idx(r):
        for m in range(SCH):
            pltpu.make_async_copy(row1_h.at[pl.ds(0, 128)], rb[r][m], isem[r]).wait()

    def fire_scat(r):
        for m in range(SCH):
            pltpu.async_copy(obuf, dacc.at[rb[r][m]], ssem[r], add=True)

    def drain_scat(r):
        for m in range(SCH):
            pltpu.make_async_copy(obuf, dacc.at[rb[r][m]], ssem[r]).wait()

    def body(g, carry):
        for q in range(NRING):
            x = g * NRING + q

            @pl.when(valid(x))
            def _(x=x, q=q):
                fire_idx(x, q)

        for q in range(NRING):
            x = g * NRING + q

            @pl.when(valid(x))
            def _(x=x, q=q):
                drain_idx(q)
                fire_scat(q)

        for q in range(NRING):
            x = g * NRING + q

            @pl.when(valid(x))
            def _(x=x, q=q):
                drain_scat(q)

        return carry

    lax.fori_loop(0, -(-DITER // NRING), body, 0)
    plsc.subcore_barrier()
    pltpu.sync_copy(dacc.at[pl.ds(s * DZ, DZ)], dbuf)
    pltpu.sync_copy(dbuf, deg_h.at[c, pl.ds(s * DZ, DZ)])


@functools.partial(
    pl.kernel,
    out_type=[
        jax.ShapeDtypeStruct((NV, DH), jnp.float32),
        jax.ShapeDtypeStruct((NV, DH), jnp.float32),
    ],
    mesh=_sc_mesh,
    compiler_params=_sc_params,
    scratch_types=[
        [[pltpu.VMEM((128,), jnp.int32) for _ in range(SCH)] for _ in range(NRING)],
        [[pltpu.VMEM((128,), jnp.int32) for _ in range(SCH)] for _ in range(NRING)],
        [pltpu.VMEM((SCH * 128, DH), jnp.float32) for _ in range(NRING)],
        pltpu.VMEM_SHARED((ACC_ROWS, DH), jnp.float32),
        [pltpu.SemaphoreType.DMA for _ in range(NRING)],
        [pltpu.SemaphoreType.DMA for _ in range(NRING)],
        [pltpu.SemaphoreType.DMA for _ in range(NRING)],
    ],
)
def _spmm_sc(col1_h, lrow_h, sa_h, sb_h, zeros_h, ta_h, tb_h,
             cb, lb, gb, acc, isem, gsem, ssem):
    c = lax.axis_index("c")
    s = lax.axis_index("s")
    lbase = c * E_PAD

    def valid(x):
        return (x >= 0) & (x * NS + s < NSCH)

    def fire_idx(x, r):
        base = (x * NS + s) * SCH * 128
        for m in range(SCH):
            pltpu.async_copy(col1_h.at[pl.ds(base + m * 128, 128)], cb[r][m], isem[r])
            pltpu.async_copy(
                lrow_h.at[pl.ds(lbase + base + m * 128, 128)], lb[r][m], isem[r]
            )

    def drain_idx(r):
        for m in range(SCH):
            pltpu.make_async_copy(col1_h.at[pl.ds(0, 128)], cb[r][m], isem[r]).wait()
            pltpu.make_async_copy(col1_h.at[pl.ds(0, 128)], lb[r][m], isem[r]).wait()

    def fire_gath(s_h, r):
        for m in range(SCH):
            pltpu.async_copy(
                s_h.at[cb[r][m]], gb[r].at[pl.ds(m * 128, 128)], gsem[r]
            )

    def drain_gath(s_h, r):
        for m in range(SCH):
            pltpu.make_async_copy(
                s_h.at[cb[r][m]], gb[r].at[pl.ds(m * 128, 128)], gsem[r]
            ).wait()

    def fire_scat(r):
        for m in range(SCH):
            pltpu.async_copy(
                gb[r].at[pl.ds(m * 128, 128)], acc.at[lb[r][m]], ssem[r], add=True
            )

    def drain_scat(r):
        for m in range(SCH):
            pltpu.make_async_copy(
                gb[r].at[pl.ds(m * 128, 128)], acc.at[lb[r][m]], ssem[r]
            ).wait()

    def copy_out(out_h, aoff, ooff, n):
        pltpu.sync_copy(acc.at[pl.ds(aoff, n)], gb[0].at[pl.ds(0, n)])
        pltpu.sync_copy(gb[0].at[pl.ds(0, n)], out_h.at[pl.ds(ooff, n)])

    for s_h, out_h in ((sa_h, ta_h), (sb_h, tb_h)):
        pltpu.sync_copy(zeros_h, gb[0])
        for q in range(6):
            pltpu.sync_copy(gb[0], acc.at[pl.ds(s * ZR + q * 256, 256)])
        pltpu.sync_copy(gb[0].at[pl.ds(0, 32)], acc.at[pl.ds(s * ZR + 1536, 32)])
        plsc.subcore_barrier()

        def body(g, carry, s_h=s_h):
            for q in range(NRING):
                x = g * NRING + q

                @pl.when(valid(x))
                def _(x=x, q=q):
                    fire_idx(x, q)

            for q in range(NRING // 2):
                x = g * NRING + q

                @pl.when(valid(x))
                def _(x=x, q=q):
                    drain_idx(q)
                    fire_gath(s_h, q)

            for q in range(NRING // 2):
                x = g * NRING + q

                @pl.when(valid(x))
                def _(x=x, q=q):
                    drain_gath(s_h, q)
                    fire_scat(q)

            for q in range(NRING // 2, NRING):
                x = g * NRING + q

                @pl.when(valid(x))
                def _(x=x, q=q):
                    drain_idx(q)
                    fire_gath(s_h, q)

            for q in range(NRING // 2, NRING):
                x = g * NRING + q

                @pl.when(valid(x))
                def _(x=x, q=q):
                    drain_gath(s_h, q)
                    fire_scat(q)

            for q in range(NRING):
                x = g * NRING + q

                @pl.when(valid(x))
                def _(x=x, q=q):
                    drain_scat(q)

            return carry

        lax.fori_loop(0, -(-NITER // NRING), body, 0)
        plsc.subcore_barrier()

        @pl.when(s < NS - 1)
        def _(out_h=out_h):
            for q in range(6):
                copy_out(out_h, s * ZR + q * 256, c * HALF + s * ZR + q * 256, 256)
            copy_out(out_h, s * ZR + 1536, c * HALF + s * ZR + 1536, 32)

        @pl.when(s == NS - 1)
        def _(out_h=out_h):
            for q in range(5):
                copy_out(out_h, (NS - 1) * ZR + q * 256,
                         c * HALF + (NS - 1) * ZR + q * 256, 256)
            copy_out(out_h, (NS - 1) * ZR + 1280,
                     c * HALF + (NS - 1) * ZR + 1280, TAIL - 1280)


_R = 5000


def _prep_tc(degp, emb0):
    def body(dref, eref, saref, sbref, diref, d2ref):
        deg = dref[0] + dref[1]
        dinv = jnp.where(deg > 0.0, lax.rsqrt(deg), 0.0)
        diref[...] = dinv
        d2ref[...] = dinv * dinv
        saref[...] = eref[:, :DH] * dinv
        sbref[...] = eref[:, DH:] * dinv

    return pl.pallas_call(
        body,
        grid=(NV // _R,),
        in_specs=[
            pl.BlockSpec((2, _R, 1), lambda i: (0, i, 0)),
            pl.BlockSpec((_R, D), lambda i: (i, 0)),
        ],
        out_specs=[
            pl.BlockSpec((_R, DH), lambda i: (i, 0)),
            pl.BlockSpec((_R, DH), lambda i: (i, 0)),
            pl.BlockSpec((_R, 1), lambda i: (i, 0)),
            pl.BlockSpec((_R, 1), lambda i: (i, 0)),
        ],
        out_shape=[
            jax.ShapeDtypeStruct((NV, DH), jnp.float32),
            jax.ShapeDtypeStruct((NV, DH), jnp.float32),
            jax.ShapeDtypeStruct((NV, 1), jnp.float32),
            jax.ShapeDtypeStruct((NV, 1), jnp.float32),
        ],
    )(degp, emb0)


def _scale_tc(ta, tb, dinv2):
    def body(taref, tbref, dref, oaref, obref):
        d = dref[...]
        oaref[...] = taref[...] * d
        obref[...] = tbref[...] * d

    return pl.pallas_call(
        body,
        grid=(NV // _R,),
        in_specs=[
            pl.BlockSpec((_R, DH), lambda i: (i, 0)),
            pl.BlockSpec((_R, DH), lambda i: (i, 0)),
            pl.BlockSpec((_R, 1), lambda i: (i, 0)),
        ],
        out_specs=[
            pl.BlockSpec((_R, DH), lambda i: (i, 0)),
            pl.BlockSpec((_R, DH), lambda i: (i, 0)),
        ],
        out_shape=[
            jax.ShapeDtypeStruct((NV, DH), jnp.float32),
            jax.ShapeDtypeStruct((NV, DH), jnp.float32),
        ],
    )(ta, tb, dinv2)


def _final_tc(emb0, tsa, tsb, dinv):
    def body(eref, a0, a1, a2, b0, b1, b2, dref, oref):
        d = dref[...]
        mix_a = d * (0.2 * a0[0] + 0.3 * a1[0] + 0.4 * a2[0])
        mix_b = d * (0.2 * b0[0] + 0.3 * b1[0] + 0.4 * b2[0])
        oref[...] = 0.1 * eref[...] + jnp.concatenate([mix_a, mix_b], axis=1)

    tspec = lambda l: pl.BlockSpec((1, _R, DH), lambda i, l=l: (l, i, 0))
    return pl.pallas_call(
        body,
        grid=(NV // _R,),
        in_specs=[pl.BlockSpec((_R, D), lambda i: (i, 0))]
        + [tspec(l) for l in range(3)] * 2
        + [pl.BlockSpec((_R, 1), lambda i: (i, 0))],
        out_specs=pl.BlockSpec((_R, D), lambda i: (i, 0)),
        out_shape=jax.ShapeDtypeStruct((NV, D), jnp.float32),
    )(emb0, tsa, tsa, tsa, tsb, tsb, tsb, dinv)


def kernel(edge_index, users_emb, items_emb):
    row = edge_index[0].astype(jnp.int32)
    col = edge_index[1].astype(jnp.int32)
    pad = E_PAD - E
    row1 = jnp.concatenate([row, jnp.full((pad,), NV, jnp.int32)])
    col1 = jnp.concatenate([col, jnp.zeros((pad,), jnp.int32)])
    trash = HALF + jnp.arange(E, dtype=jnp.int32) % (ACC_ROWS - HALF)
    padt = HALF + jnp.arange(pad, dtype=jnp.int32) % (ACC_ROWS - HALF)
    lrow = jnp.concatenate(
        [
            jnp.where(row < HALF, row, trash), padt,
            jnp.where(row >= HALF, row - HALF, trash), padt,
        ]
    )
    emb0 = jnp.concatenate([users_emb, items_emb], axis=0)
    zeros2 = jnp.zeros((SCH * 128, DH), jnp.float32)
    zd = jnp.zeros((DZ,), jnp.float32)

    degp = _deg_sc(row1, zd)
    degp = degp[:, :NV].reshape(NC, NV, 1)
    s0a, s0b, dinv, dinv2 = _prep_tc(degp, emb0)

    def layer(s, _):
        sa, sb = s
        ta, tb = _spmm_sc(col1, lrow, sa, sb, zeros2)
        return _scale_tc(ta, tb, dinv2), (ta, tb)

    _, (tsa, tsb) = lax.scan(layer, (s0a, s0b), None, length=3)
    final = _final_tc(emb0, tsa, tsb, dinv)

    uK = final[:NUM_USERS]
    iK = final[NUM_USERS:]
    return (uK, users_emb, iK, items_emb, uK, iK, uK, iK)


# speedup vs baseline: 12.7677x; 1.0422x over previous
import functools

import jax
import jax.numpy as jnp
from jax import lax
from jax.experimental import pallas as pl
from jax.experimental.pallas import tpu as pltpu
from jax.experimental.pallas import tpu_sc as plsc

NUM_USERS = 10000
NUM_ITEMS = 40000
NV = NUM_USERS + NUM_ITEMS
E = 800000
D = 64
DH = 32
NC, NS = 2, 16
NW = NC * NS
HALF = NV // NC

SCH = 2
ER = 6256
E_PAD = ER * 128
NSCH = ER // SCH
NITER = -(-NSCH // NS)
NRING = 8
ACC_ROWS = 25088
ZR = ACC_ROWS // NS
TAIL = HALF - 15 * ZR
DEG_PAD = 50048
DZ = DEG_PAD // NS
DITER = -(-NSCH // NW)

_GDN = lax.GatherDimensionNumbers(
    offset_dims=(), collapsed_slice_dims=(0,), start_index_map=(0,)
)


def _bcast_lane(vec, j):
    idx = jnp.full((16, 1), j, jnp.int32)
    return lax.gather(vec, idx, _GDN, slice_sizes=(1,),
                      mode=lax.GatherScatterMode.PROMISE_IN_BOUNDS)


_sc_mesh = plsc.VectorSubcoreMesh(core_axis_name="c", subcore_axis_name="s")
_sc_params = pltpu.CompilerParams(use_tc_tiling_on_sc=False)


@functools.partial(
    pl.kernel,
    out_type=jax.ShapeDtypeStruct((NC, DEG_PAD), jnp.float32),
    mesh=_sc_mesh,
    compiler_params=_sc_params,
    scratch_types=[
        [[pltpu.VMEM((128,), jnp.int32) for _ in range(SCH)] for _ in range(NRING)],
        pltpu.VMEM((128,), jnp.float32),
        pltpu.VMEM((DZ,), jnp.float32),
        pltpu.VMEM_SHARED((DEG_PAD,), jnp.float32),
        [pltpu.SemaphoreType.DMA for _ in range(NRING)],
        [pltpu.SemaphoreType.DMA for _ in range(NRING)],
    ],
)
def _deg_sc(row1_h, zd_h, deg_h, rb, obuf, dbuf, dacc, isem, ssem):
    c = lax.axis_index("c")
    s = lax.axis_index("s")
    w = s * NC + c

    for m in range(8):
        obuf[pl.ds(m * 16, 16)] = jnp.full((16,), 1.0, jnp.float32)
    pltpu.sync_copy(zd_h, dbuf)
    pltpu.sync_copy(dbuf, dacc.at[pl.ds(s * DZ, DZ)])
    plsc.subcore_barrier()

    def valid(x):
        return (x >= 0) & (x * NW + w < NSCH)

    def fire_idx(x, r):
        base = (x * NW + w) * SCH * 128
        for m in range(SCH):
            pltpu.async_copy(row1_h.at[pl.ds(base + m * 128, 128)], rb[r][m], isem[r])

    def drain_idx(r):
        for m in range(SCH):
            pltpu.make_async_copy(row1_h.at[pl.ds(0, 128)], rb[r][m], isem[r]).wait()

    def fire_scat(r):
        for m in range(SCH):
            pltpu.async_copy(obuf, dacc.at[rb[r][m]], ssem[r], add=True)

    def drain_scat(r):
        for m in range(SCH):
            pltpu.make_async_copy(obuf, dacc.at[rb[r][m]], ssem[r]).wait()

    def body(g, carry):
        for q in range(NRING):
            x = g * NRING + q

            @pl.when(valid(x))
            def _(x=x, q=q):
                fire_idx(x, q)

        for q in range(NRING):
            x = g * NRING + q

            @pl.when(valid(x))
            def _(x=x, q=q):
                drain_idx(q)
                fire_scat(q)

        for q in range(NRING):
            x = g * NRING + q

            @pl.when(valid(x))
            def _(x=x, q=q):
                drain_scat(q)

        return carry

    lax.fori_loop(0, -(-DITER // NRING), body, 0)
    plsc.subcore_barrier()
    pltpu.sync_copy(dacc.at[pl.ds(s * DZ, DZ)], dbuf)
    pltpu.sync_copy(dbuf, deg_h.at[c, pl.ds(s * DZ, DZ)])


@functools.partial(
    pl.kernel,
    out_type=[
        jax.ShapeDtypeStruct((NV, DH), jnp.float32),
        jax.ShapeDtypeStruct((NV, DH), jnp.float32),
        jax.ShapeDtypeStruct((NV, DH), jnp.float32),
        jax.ShapeDtypeStruct((NV, DH), jnp.float32),
    ],
    mesh=_sc_mesh,
    compiler_params=_sc_params,
    scratch_types=[
        [[pltpu.VMEM((128,), jnp.int32) for _ in range(SCH)] for _ in range(NRING)],
        [[pltpu.VMEM((128,), jnp.int32) for _ in range(SCH)] for _ in range(NRING)],
        [pltpu.VMEM((SCH * 128, DH), jnp.float32) for _ in range(NRING)],
        pltpu.VMEM((256, DH), jnp.float32),
        pltpu.VMEM((ZR,), jnp.float32),
        pltpu.VMEM_SHARED((ACC_ROWS, DH), jnp.float32),
        [pltpu.SemaphoreType.DMA for _ in range(NRING)],
        [pltpu.SemaphoreType.DMA for _ in range(NRING)],
        [pltpu.SemaphoreType.DMA for _ in range(NRING)],
    ],
)
def _spmm_sc(col1_h, lrow_h, sa_h, sb_h, zeros_h, d2_h,
             ta_h, tb_h, na_h, nb_h,
             cb, lb, gb, sb2, dbuf, acc, isem, gsem, ssem):
    c = lax.axis_index("c")
    s = lax.axis_index("s")
    lbase = c * E_PAD

    def valid(x):
        return (x >= 0) & (x * NS + s < NSCH)

    def fire_idx(x, r):
        base = (x * NS + s) * SCH * 128
        for m in range(SCH):
            pltpu.async_copy(col1_h.at[pl.ds(base + m * 128, 128)], cb[r][m], isem[r])
            pltpu.async_copy(
                lrow_h.at[pl.ds(lbase + base + m * 128, 128)], lb[r][m], isem[r]
            )

    def drain_idx(r):
        for m in range(SCH):
            pltpu.make_async_copy(col1_h.at[pl.ds(0, 128)], cb[r][m], isem[r]).wait()
            pltpu.make_async_copy(col1_h.at[pl.ds(0, 128)], lb[r][m], isem[r]).wait()

    def fire_gath(s_h, r):
        for m in range(SCH):
            pltpu.async_copy(
                s_h.at[cb[r][m]], gb[r].at[pl.ds(m * 128, 128)], gsem[r]
            )

    def drain_gath(s_h, r):
        for m in range(SCH):
            pltpu.make_async_copy(
                s_h.at[cb[r][m]], gb[r].at[pl.ds(m * 128, 128)], gsem[r]
            ).wait()

    def fire_scat(r):
        for m in range(SCH):
            pltpu.async_copy(
                gb[r].at[pl.ds(m * 128, 128)], acc.at[lb[r][m]], ssem[r], add=True
            )

    def drain_scat(r):
        for m in range(SCH):
            pltpu.make_async_copy(
                gb[r].at[pl.ds(m * 128, 128)], acc.at[lb[r][m]], ssem[r]
            ).wait()

    pltpu.sync_copy(d2_h.at[pl.ds(c * HALF + s * ZR, ZR)], dbuf)

    def copy_out(t_h, n_h, aoff, ooff, n):
        pltpu.sync_copy(acc.at[pl.ds(aoff, n)], gb[0].at[pl.ds(0, n)])
        pltpu.sync_copy(gb[0].at[pl.ds(0, n)], t_h.at[pl.ds(ooff, n)])

        def grp(g, carry):
            dv = dbuf[pl.ds(aoff - s * ZR + g * 16, 16)]
            for j in range(16):
                mlt = _bcast_lane(dv, j)
                r = g * 16 + j
                sb2[r, pl.ds(0, 16)] = gb[0][r, pl.ds(0, 16)] * mlt
                sb2[r, pl.ds(16, 16)] = gb[0][r, pl.ds(16, 16)] * mlt
            return carry

        lax.fori_loop(0, n // 16, grp, 0)
        @pl.when((n % 16) > 0)
        def _():
            g0 = n // 16
            dv = dbuf[pl.ds(aoff - s * ZR + g0 * 16, 16)]
            for j in range(n % 16):
                mlt = _bcast_lane(dv, j)
                r = g0 * 16 + j
                sb2[r, pl.ds(0, 16)] = gb[0][r, pl.ds(0, 16)] * mlt
                sb2[r, pl.ds(16, 16)] = gb[0][r, pl.ds(16, 16)] * mlt
        pltpu.sync_copy(sb2.at[pl.ds(0, n)], n_h.at[pl.ds(ooff, n)])

    for s_h, out_h in ((sa_h, (ta_h, na_h)), (sb_h, (tb_h, nb_h))):
        pltpu.sync_copy(zeros_h, gb[0])
        for q in range(6):
            pltpu.sync_copy(gb[0], acc.at[pl.ds(s * ZR + q * 256, 256)])
        pltpu.sync_copy(gb[0].at[pl.ds(0, 32)], acc.at[pl.ds(s * ZR + 1536, 32)])
        plsc.subcore_barrier()

        def body(g, carry, s_h=s_h):
            for q in range(NRING):
                x = g * NRING + q

                @pl.when(valid(x))
                def _(x=x, q=q):
                    fire_idx(x, q)

            for q in range(NRING // 2):
                x = g * NRING + q

                @pl.when(valid(x))
                def _(x=x, q=q):
                    drain_idx(q)
                    fire_gath(s_h, q)

            for q in range(NRING // 2):
                x = g * NRING + q

                @pl.when(valid(x))
                def _(x=x, q=q):
                    drain_gath(s_h, q)
                    fire_scat(q)

            for q in range(NRING // 2, NRING):
                x = g * NRING + q

                @pl.when(valid(x))
                def _(x=x, q=q):
                    drain_idx(q)
                    fire_gath(s_h, q)

            for q in range(NRING // 2, NRING):
                x = g * NRING + q

                @pl.when(valid(x))
                def _(x=x, q=q):
                    drain_gath(s_h, q)
                    fire_scat(q)

            for q in range(NRING):
                x = g * NRING + q

                @pl.when(valid(x))
                def _(x=x, q=q):
                    drain_scat(q)

            return carry

        lax.fori_loop(0, -(-NITER // NRING), body, 0)
        plsc.subcore_barrier()

        t_h, n_h = out_h

        @pl.when(s < NS - 1)
        def _(t_h=t_h, n_h=n_h):
            for q in range(6):
                copy_out(t_h, n_h, s * ZR + q * 256, c * HALF + s * ZR + q * 256, 256)
            copy_out(t_h, n_h, s * ZR + 1536, c * HALF + s * ZR + 1536, 32)

        @pl.when(s == NS - 1)
        def _(t_h=t_h, n_h=n_h):
            for q in range(5):
                copy_out(t_h, n_h, (NS - 1) * ZR + q * 256,
                         c * HALF + (NS - 1) * ZR + q * 256, 256)
            copy_out(t_h, n_h, (NS - 1) * ZR + 1280,
                     c * HALF + (NS - 1) * ZR + 1280, TAIL - 1280)


_R = 5000


def _prep_tc(degp, emb0):
    def body(dref, eref, saref, sbref, diref, d2ref):
        deg = dref[0] + dref[1]
        dinv = jnp.where(deg > 0.0, lax.rsqrt(deg), 0.0)
        diref[...] = dinv
        d2ref[...] = dinv * dinv
        saref[...] = eref[:, :DH] * dinv
        sbref[...] = eref[:, DH:] * dinv

    return pl.pallas_call(
        body,
        grid=(NV // _R,),
        in_specs=[
            pl.BlockSpec((2, _R, 1), lambda i: (0, i, 0)),
            pl.BlockSpec((_R, D), lambda i: (i, 0)),
        ],
        out_specs=[
            pl.BlockSpec((_R, DH), lambda i: (i, 0)),
            pl.BlockSpec((_R, DH), lambda i: (i, 0)),
            pl.BlockSpec((_R, 1), lambda i: (i, 0)),
            pl.BlockSpec((_R, 1), lambda i: (i, 0)),
        ],
        out_shape=[
            jax.ShapeDtypeStruct((NV, DH), jnp.float32),
            jax.ShapeDtypeStruct((NV, DH), jnp.float32),
            jax.ShapeDtypeStruct((NV, 1), jnp.float32),
            jax.ShapeDtypeStruct((NV, 1), jnp.float32),
        ],
    )(degp, emb0)


def _final_tc(emb0, tsa, tsb, dinv):
    def body(eref, a0, a1, a2, b0, b1, b2, dref, oref):
        d = dref[...]
        mix_a = d * (0.2 * a0[0] + 0.3 * a1[0] + 0.4 * a2[0])
        mix_b = d * (0.2 * b0[0] + 0.3 * b1[0] + 0.4 * b2[0])
        oref[...] = 0.1 * eref[...] + jnp.concatenate([mix_a, mix_b], axis=1)

    tspec = lambda l: pl.BlockSpec((1, _R, DH), lambda i, l=l: (l, i, 0))
    return pl.pallas_call(
        body,
        grid=(NV // _R,),
        in_specs=[pl.BlockSpec((_R, D), lambda i: (i, 0))]
        + [tspec(l) for l in range(3)] * 2
        + [pl.BlockSpec((_R, 1), lambda i: (i, 0))],
        out_specs=pl.BlockSpec((_R, D), lambda i: (i, 0)),
        out_shape=jax.ShapeDtypeStruct((NV, D), jnp.float32),
    )(emb0, tsa, tsa, tsa, tsb, tsb, tsb, dinv)


def kernel(edge_index, users_emb, items_emb):
    row = edge_index[0].astype(jnp.int32)
    col = edge_index[1].astype(jnp.int32)
    pad = E_PAD - E
    row1 = jnp.concatenate([row, jnp.full((pad,), NV, jnp.int32)])
    col1 = jnp.concatenate([col, jnp.zeros((pad,), jnp.int32)])
    trash = HALF + jnp.arange(E, dtype=jnp.int32) % (ACC_ROWS - HALF)
    padt = HALF + jnp.arange(pad, dtype=jnp.int32) % (ACC_ROWS - HALF)
    lrow = jnp.concatenate(
        [
            jnp.where(row < HALF, row, trash), padt,
            jnp.where(row >= HALF, row - HALF, trash), padt,
        ]
    )
    emb0 = jnp.concatenate([users_emb, items_emb], axis=0)
    zeros2 = jnp.zeros((SCH * 128, DH), jnp.float32)
    zd = jnp.zeros((DZ,), jnp.float32)

    degp = _deg_sc(row1, zd)
    degp = degp[:, :NV].reshape(NC, NV, 1)
    s0a, s0b, dinv, dinv2 = _prep_tc(degp, emb0)
    d2f = jnp.concatenate([dinv2.reshape(NV), jnp.zeros((176,), jnp.float32)])

    def layer(s, _):
        sa, sb = s
        ta, tb, na, nb = _spmm_sc(col1, lrow, sa, sb, zeros2, d2f)
        return (na, nb), (ta, tb)

    _, (tsa, tsb) = lax.scan(layer, (s0a, s0b), None, length=3)
    final = _final_tc(emb0, tsa, tsb, dinv)

    uK = final[:NUM_USERS]
    iK = final[NUM_USERS:]
    return (uK, users_emb, iK, items_emb, uK, iK, uK, iK)
